# baseline (speedup 1.0000x reference)
import functools

import jax
import jax.numpy as jnp
from jax import lax
from jax.experimental import pallas as pl
from jax.experimental.pallas import tpu as pltpu
from jax.experimental.pallas import tpu_sc as plsc

B = 32
N = 512
K = 20
R = B * N * K
DP = 128

_SC_CORES = 2
_SC_SUBCORES = 16
_NW = _SC_CORES * _SC_SUBCORES
_CH = 256


def _elu(x):
    return jnp.where(x > 0, x, jnp.exp(jnp.minimum(x, 0.0)) - 1.0)


def _dot(a, b):
    return lax.dot_general(a, b, (((1,), (0,)), ((), ())),
                           precision=lax.Precision.DEFAULT)


def _knn_body(boff, xcb_ref, sqc_ref, sqr_ref, idx_ref):
    b = pl.program_id(0) + boff
    x = xcb_ref[0]
    xx = lax.dot_general(x, x, (((1,), (1,)), ((), ())),
                         precision=lax.Precision.DEFAULT)
    d2 = (sqc_ref[0] + sqr_ref[0]) - 2.0 * xx

    cols = lax.broadcasted_iota(jnp.int32, (N, N), 1)
    sels = []
    d2w = d2
    for _ in range(K):
        sel = jnp.argmin(d2w, axis=1)[:, None]
        sels.append(sel)
        d2w = jnp.where(cols == sel, jnp.float32(jnp.inf), d2w)
    idx_ref[0] = jnp.concatenate(sels, axis=1) + b * N


def _knn(xcb, sqc, sqr, boff):
    d = xcb.shape[-1]
    bh = xcb.shape[0]
    return pl.pallas_call(
        functools.partial(_knn_body, boff),
        grid=(bh,),
        in_specs=[
            pl.BlockSpec((1, N, d), lambda b: (b, 0, 0)),
            pl.BlockSpec((1, N, 1), lambda b: (b, 0, 0)),
            pl.BlockSpec((1, 1, N), lambda b: (b, 0, 0)),
        ],
        out_specs=pl.BlockSpec((1, N, K), lambda b: (b, 0, 0)),
        out_shape=jax.ShapeDtypeStruct((bh, N, K), jnp.int32),
    )(xcb, sqc, sqr)


@functools.cache
def _sc_gather(r):
    per_w = r // _NW
    n_it = per_w // _CH
    mesh = plsc.VectorSubcoreMesh(core_axis_name="c", subcore_axis_name="s")

    @functools.partial(
        pl.kernel,
        mesh=mesh,
        out_type=jax.ShapeDtypeStruct((r, DP), jnp.float32),
        scratch_types=[
            pltpu.VMEM((_CH,), jnp.int32),
            pltpu.VMEM((_CH,), jnp.int32),
            pltpu.VMEM((_CH, DP), jnp.float32),
            pltpu.VMEM((_CH, DP), jnp.float32),
            pltpu.SemaphoreType.DMA,
            pltpu.SemaphoreType.DMA,
        ],
    )
    def k(table_hbm, idx_hbm, out_hbm, idx0, idx1, rows0, rows1, sem0, sem1):
        wid = lax.axis_index("s") * _SC_CORES + lax.axis_index("c")
        base = wid * per_w
        idx_b, rows_b, sem_b = (idx0, idx1), (rows0, rows1), (sem0, sem1)

        pltpu.sync_copy(idx_hbm.at[pl.ds(base, _CH)], idx0)
        pltpu.async_copy(table_hbm.at[idx0], rows0, sem0)

        def body(j, carry):
            for b in (0, 1):
                i = 2 * j + b
                nb = 1 - b

                @pl.when(i + 1 < n_it)
                def _():
                    off_n = base + (i + 1) * _CH
                    pltpu.sync_copy(idx_hbm.at[pl.ds(off_n, _CH)], idx_b[nb])
                    pltpu.async_copy(table_hbm.at[idx_b[nb]], rows_b[nb],
                                     sem_b[nb])

                pltpu.make_async_copy(table_hbm.at[idx_b[b]], rows_b[b],
                                      sem_b[b]).wait()
                pltpu.sync_copy(rows_b[b],
                                out_hbm.at[pl.ds(base + i * _CH, _CH)])
            return carry

        lax.fori_loop(0, n_it // 2, body, 0)

    return k


_NCH = 128


def _edge_mlp_body(d, xcb_ref, g_ref, w1_ref, b1_ref, w2_ref, b2_ref, o_ref):
    xi = xcb_ref[0]
    xj = g_ref[0][:, :d]
    xir = jnp.broadcast_to(xi[:, None, :], (_NCH, K, d)).reshape(_NCH * K, d)
    e = jnp.concatenate([xir, xj - xir], axis=1)
    h = _elu(_dot(e, w1_ref[...]) + b1_ref[...])
    o = _dot(h, w2_ref[...]) + b2_ref[...]
    o = jnp.max(o.reshape(_NCH, K, 64), axis=1)
    o_ref[0] = _elu(o)


def _edge_mlp(xcb, g3, w1, b1, w2, b2):
    d = xcb.shape[-1]
    dh = w1.shape[1]
    bh = xcb.shape[0]
    return pl.pallas_call(
        functools.partial(_edge_mlp_body, d),
        grid=(bh, N // _NCH),
        in_specs=[
            pl.BlockSpec((1, _NCH, d), lambda b, j: (b, j, 0)),
            pl.BlockSpec((1, _NCH * K, DP), lambda b, j: (b, j, 0)),
            pl.BlockSpec((2 * d, dh), lambda b, j: (0, 0)),
            pl.BlockSpec((1, dh), lambda b, j: (0, 0)),
            pl.BlockSpec((dh, 64), lambda b, j: (0, 0)),
            pl.BlockSpec((1, 64), lambda b, j: (0, 0)),
        ],
        out_specs=pl.BlockSpec((1, _NCH, 64), lambda b, j: (b, j, 0)),
        out_shape=jax.ShapeDtypeStruct((bh, N, 64), jnp.float32),
    )(xcb, g3, w1, b1, w2, b2)


def _ffn_body(x_ref, c_ref, w1_ref, b1_ref, w2_ref, b2_ref, w3_ref, b3_ref,
              o_ref):
    pooled = jnp.mean(x_ref[...], axis=1)
    g = jnp.concatenate([pooled, c_ref[...]], axis=1)
    h = _elu(_dot(g, w1_ref[...]) + b1_ref[...])
    h = _elu(_dot(h, w2_ref[...]) + b2_ref[...])
    o_ref[...] = _dot(h, w3_ref[...]) + b3_ref[...]


def _ffn(x3, c, w1, b1, w2, b2, w3, b3):
    return pl.pallas_call(
        _ffn_body,
        out_shape=jax.ShapeDtypeStruct((B, 1), jnp.float32),
    )(x3, c, w1, b1[None], w2, b2[None], w3, b3[None])


_BH = 16


def _conv_layer_half(xh, cbh, W1, b1, W2, b2):
    xcb = jnp.concatenate([xh, cbh], axis=-1)
    d = xcb.shape[-1]
    sq = jnp.sum(xcb * xcb, axis=-1)
    table = jnp.pad(xcb, ((0, 0), (0, 0), (0, DP - d))).reshape(_BH * N, DP)
    rh = _BH * N * K
    idx = _knn(xcb, sq[:, :, None], sq[:, None, :], 0)
    g = _sc_gather(rh)(table, idx.reshape(rh))
    return _edge_mlp(xcb, g.reshape(_BH, N * K, DP), W1, b1[None], W2, b2[None])


def kernel(pos, y, batch, conv0_W1, conv0_b1, conv0_W2, conv0_b2,
           conv1_W1, conv1_b1, conv1_W2, conv1_b2,
           conv2_W1, conv2_b1, conv2_W2, conv2_b2,
           ffn_W1, ffn_b1, ffn_W2, ffn_b2, ffn_W3, ffn_b3):
    c = jax.nn.one_hot(y, 10, dtype=jnp.float32)
    cb3 = jnp.broadcast_to(c[:, None, :], (B, N, 10))
    x = jnp.reshape(pos, (B, N, 3))
    halves = [x[:_BH], x[_BH:]]
    cbs = [cb3[:_BH], cb3[_BH:]]
    for (W1, b1, W2, b2) in (
            (conv0_W1, conv0_b1, conv0_W2, conv0_b2),
            (conv1_W1, conv1_b1, conv1_W2, conv1_b2),
            (conv2_W1, conv2_b1, conv2_W2, conv2_b2)):
        halves = [_conv_layer_half(halves[h], cbs[h], W1, b1, W2, b2)
                  for h in range(2)]
    x = jnp.concatenate(halves, axis=0)
    return _ffn(x, c, ffn_W1, ffn_b1, ffn_W2, ffn_b2, ffn_W3, ffn_b3)


# speedup vs baseline: 1.1670x; 1.1670x over previous
import functools

import jax
import jax.numpy as jnp
from jax import lax
from jax.experimental import pallas as pl
from jax.experimental.pallas import tpu as pltpu
from jax.experimental.pallas import tpu_sc as plsc

B = 32
N = 512
K = 20
R = B * N * K
DP = 128

_SC_CORES = 2
_SC_SUBCORES = 16
_NW = _SC_CORES * _SC_SUBCORES
_CH = 256


def _elu(x):
    return jnp.where(x > 0, x, jnp.exp(jnp.minimum(x, 0.0)) - 1.0)


def _dot(a, b):
    return lax.dot_general(a, b, (((1,), (0,)), ((), ())),
                           precision=lax.Precision.DEFAULT)


def _knn_body(boff, xcb_ref, sqc_ref, sqr_ref, idx_ref):
    b = pl.program_id(0) + boff
    x = xcb_ref[0]
    xx = lax.dot_general(x, x, (((1,), (1,)), ((), ())),
                         precision=lax.Precision.DEFAULT)
    d2 = (sqc_ref[0] + sqr_ref[0]) - 2.0 * xx

    cols = lax.broadcasted_iota(jnp.int32, (N, N), 1)
    sels = []
    d2w = d2
    for _ in range(K):
        sel = jnp.argmin(d2w, axis=1)[:, None]
        sels.append(sel)
        d2w = jnp.where(cols == sel, jnp.float32(jnp.inf), d2w)
    idx_ref[0] = jnp.concatenate(sels, axis=1) + b * N


def _knn(xcb, sqc, sqr, boff):
    d = xcb.shape[-1]
    bh = xcb.shape[0]
    return pl.pallas_call(
        functools.partial(_knn_body, boff),
        grid=(bh,),
        in_specs=[
            pl.BlockSpec((1, N, d), lambda b: (b, 0, 0)),
            pl.BlockSpec((1, N, 1), lambda b: (b, 0, 0)),
            pl.BlockSpec((1, 1, N), lambda b: (b, 0, 0)),
        ],
        out_specs=pl.BlockSpec((1, N, K), lambda b: (b, 0, 0)),
        out_shape=jax.ShapeDtypeStruct((bh, N, K), jnp.int32),
    )(xcb, sqc, sqr)


@functools.cache
def _sc_gather(r):
    per_w = r // _NW
    n_it = per_w // _CH
    mesh = plsc.VectorSubcoreMesh(core_axis_name="c", subcore_axis_name="s")

    @functools.partial(
        pl.kernel,
        mesh=mesh,
        out_type=jax.ShapeDtypeStruct((r, DP), jnp.float32),
        scratch_types=[
            pltpu.VMEM((_CH,), jnp.int32),
            pltpu.VMEM((_CH,), jnp.int32),
            pltpu.VMEM((_CH, DP), jnp.float32),
            pltpu.VMEM((_CH, DP), jnp.float32),
            pltpu.SemaphoreType.DMA,
            pltpu.SemaphoreType.DMA,
        ],
    )
    def k(table_hbm, idx_hbm, out_hbm, idx0, idx1, rows0, rows1, sem0, sem1):
        wid = lax.axis_index("s") * _SC_CORES + lax.axis_index("c")
        base = wid * per_w
        idx_b, rows_b, sem_b = (idx0, idx1), (rows0, rows1), (sem0, sem1)

        pltpu.sync_copy(idx_hbm.at[pl.ds(base, _CH)], idx0)
        pltpu.async_copy(table_hbm.at[idx0], rows0, sem0)

        def body(j, carry):
            for b in (0, 1):
                i = 2 * j + b
                nb = 1 - b

                @pl.when(i + 1 < n_it)
                def _():
                    off_n = base + (i + 1) * _CH
                    pltpu.sync_copy(idx_hbm.at[pl.ds(off_n, _CH)], idx_b[nb])
                    pltpu.async_copy(table_hbm.at[idx_b[nb]], rows_b[nb],
                                     sem_b[nb])

                pltpu.make_async_copy(table_hbm.at[idx_b[b]], rows_b[b],
                                      sem_b[b]).wait()
                pltpu.sync_copy(rows_b[b],
                                out_hbm.at[pl.ds(base + i * _CH, _CH)])
            return carry

        lax.fori_loop(0, n_it // 2, body, 0)

    return k


_NCH = 512


def _edge_mlp_body(d, xcb_ref, g_ref, w1_ref, b1_ref, w2_ref, b2_ref, o_ref):
    xi = xcb_ref[0]
    xj = g_ref[0][:, :d]
    xir = jnp.broadcast_to(xi[:, None, :], (_NCH, K, d)).reshape(_NCH * K, d)
    e = jnp.concatenate([xir, xj - xir], axis=1)
    h = _elu(_dot(e, w1_ref[...]) + b1_ref[...])
    o = _dot(h, w2_ref[...]) + b2_ref[...]
    o = jnp.max(o.reshape(_NCH, K, 64), axis=1)
    o_ref[0] = _elu(o)


def _edge_mlp(xcb, g3, w1, b1, w2, b2):
    d = xcb.shape[-1]
    dh = w1.shape[1]
    bh = xcb.shape[0]
    return pl.pallas_call(
        functools.partial(_edge_mlp_body, d),
        grid=(bh, N // _NCH),
        in_specs=[
            pl.BlockSpec((1, _NCH, d), lambda b, j: (b, j, 0)),
            pl.BlockSpec((1, _NCH * K, DP), lambda b, j: (b, j, 0)),
            pl.BlockSpec((2 * d, dh), lambda b, j: (0, 0)),
            pl.BlockSpec((1, dh), lambda b, j: (0, 0)),
            pl.BlockSpec((dh, 64), lambda b, j: (0, 0)),
            pl.BlockSpec((1, 64), lambda b, j: (0, 0)),
        ],
        out_specs=pl.BlockSpec((1, _NCH, 64), lambda b, j: (b, j, 0)),
        out_shape=jax.ShapeDtypeStruct((bh, N, 64), jnp.float32),
    )(xcb, g3, w1, b1, w2, b2)


def _ffn_body(x_ref, c_ref, w1_ref, b1_ref, w2_ref, b2_ref, w3_ref, b3_ref,
              o_ref):
    pooled = jnp.mean(x_ref[...], axis=1)
    g = jnp.concatenate([pooled, c_ref[...]], axis=1)
    h = _elu(_dot(g, w1_ref[...]) + b1_ref[...])
    h = _elu(_dot(h, w2_ref[...]) + b2_ref[...])
    o_ref[...] = _dot(h, w3_ref[...]) + b3_ref[...]


def _ffn(x3, c, w1, b1, w2, b2, w3, b3):
    return pl.pallas_call(
        _ffn_body,
        out_shape=jax.ShapeDtypeStruct((B, 1), jnp.float32),
    )(x3, c, w1, b1[None], w2, b2[None], w3, b3[None])


_BH = 16


def _conv_layer(x3, cb3, W1, b1, W2, b2):
    xcb = jnp.concatenate([x3, cb3], axis=-1)
    d = xcb.shape[-1]
    sq = jnp.sum(xcb * xcb, axis=-1)
    table = jnp.pad(xcb, ((0, 0), (0, 0), (0, DP - d))).reshape(B * N, DP)
    rh = _BH * N * K
    outs = []
    for h in range(B // _BH):
        s = slice(h * _BH, (h + 1) * _BH)
        idx = _knn(xcb[s], sq[s, :, None], sq[s, None, :], h * _BH)
        g = _sc_gather(rh)(table, idx.reshape(rh))
        outs.append(_edge_mlp(xcb[s], g.reshape(_BH, N * K, DP),
                              W1, b1[None], W2, b2[None]))
    return jnp.concatenate(outs, axis=0)


def kernel(pos, y, batch, conv0_W1, conv0_b1, conv0_W2, conv0_b2,
           conv1_W1, conv1_b1, conv1_W2, conv1_b2,
           conv2_W1, conv2_b1, conv2_W2, conv2_b2,
           ffn_W1, ffn_b1, ffn_W2, ffn_b2, ffn_W3, ffn_b3):
    c = jax.nn.one_hot(y, 10, dtype=jnp.float32)
    cb3 = jnp.broadcast_to(c[:, None, :], (B, N, 10))
    x = jnp.reshape(pos, (B, N, 3))
    x = _conv_layer(x, cb3, conv0_W1, conv0_b1, conv0_W2, conv0_b2)
    x = _conv_layer(x, cb3, conv1_W1, conv1_b1, conv1_W2, conv1_b2)
    x = _conv_layer(x, cb3, conv2_W1, conv2_b1, conv2_W2, conv2_b2)
    return _ffn(x, c, ffn_W1, ffn_b1, ffn_W2, ffn_b2, ffn_W3, ffn_b3)
